# Initial kernel scaffold; baseline (speedup 1.0000x reference)
import jax
import jax.numpy as jnp
from jax.experimental import pallas as pl


def kernel(xyz):
    raise NotImplementedError("write your pallas kernel here")


# speedup vs baseline: 8.4900x; 8.4900x over previous
import functools

import jax
import jax.numpy as jnp
from jax import lax
from jax.experimental import pallas as pl
from jax.experimental.pallas import tpu as pltpu

_N = 8192
_NQ = 1024
_K = 16
_QB = 128


def _fps_kernel(xyz_ref, ptsT_ref):
    xs = xyz_ref[0, 0, :].reshape(64, 128)
    ys = xyz_ref[0, 1, :].reshape(64, 128)
    zs = xyz_ref[0, 2, :].reshape(64, 128)
    iota_n = (lax.broadcasted_iota(jnp.int32, (64, 128), 0) * 128
              + lax.broadcasted_iota(jnp.int32, (64, 128), 1))
    iota_q = (lax.broadcasted_iota(jnp.int32, (8, 128), 0) * 128
              + lax.broadcasted_iota(jnp.int32, (8, 128), 1))

    def body(i, carry):
        dists, far, cxv, cyv, czv = carry
        qmask = iota_q == i
        nmask = iota_n == far
        fx = jnp.sum(jnp.where(nmask, xs, 0.0))
        fy = jnp.sum(jnp.where(nmask, ys, 0.0))
        fz = jnp.sum(jnp.where(nmask, zs, 0.0))
        cxv = jnp.where(qmask, fx, cxv)
        cyv = jnp.where(qmask, fy, cyv)
        czv = jnp.where(qmask, fz, czv)
        dx = xs - fx
        dy = ys - fy
        dz = zs - fz
        d = dx * dx + dy * dy + dz * dz
        dists = jnp.minimum(dists, d)
        m = jnp.max(dists)
        far = jnp.min(jnp.where(dists == m, iota_n, jnp.int32(2**30)))
        return dists, far, cxv, cyv, czv

    init = (jnp.full((64, 128), 1e10, dtype=jnp.float32),
            jnp.int32(0),
            jnp.zeros((8, 128), dtype=jnp.float32),
            jnp.zeros((8, 128), dtype=jnp.float32),
            jnp.zeros((8, 128), dtype=jnp.float32))
    _, _, cxv, cyv, czv = lax.fori_loop(0, _NQ, body, init)
    ptsT_ref[0, 0, :] = cxv.reshape(_NQ)
    ptsT_ref[0, 1, :] = cyv.reshape(_NQ)
    ptsT_ref[0, 2, :] = czv.reshape(_NQ)


def _knn_kernel(xyz_ref, ptsT_ref, out_ref):
    xs = xyz_ref[0, 0, :][None, :]
    ys = xyz_ref[0, 1, :][None, :]
    zs = xyz_ref[0, 2, :][None, :]
    qx = ptsT_ref[0, 0, :][:, None]
    qy = ptsT_ref[0, 1, :][:, None]
    qz = ptsT_ref[0, 2, :][:, None]
    x_sq = xs * xs + ys * ys + zs * zs
    q_sq = qx * qx + qy * qy + qz * qz
    bxs = xs.astype(jnp.bfloat16).astype(jnp.float32)
    bys = ys.astype(jnp.bfloat16).astype(jnp.float32)
    bzs = zs.astype(jnp.bfloat16).astype(jnp.float32)
    bqx = qx.astype(jnp.bfloat16).astype(jnp.float32)
    bqy = qy.astype(jnp.bfloat16).astype(jnp.float32)
    bqz = qz.astype(jnp.bfloat16).astype(jnp.float32)
    dot = bqx * bxs + bqy * bys + bqz * bzs
    d2 = (q_sq + x_sq) - 2.0 * dot
    iota_n = lax.broadcasted_iota(jnp.int32, (_QB, _N), 1)
    for k in range(_K):
        m = jnp.min(d2, axis=1, keepdims=True)
        cand = jnp.where(d2 == m, iota_n, jnp.int32(2**30))
        i_k = jnp.min(cand, axis=1)
        out_ref[0, k, :] = i_k
        d2 = jnp.where(iota_n == i_k[:, None], jnp.float32(jnp.inf), d2)


@jax.jit
def kernel(xyz):
    b = xyz.shape[0]
    xyzT = jnp.transpose(xyz, (0, 2, 1))

    ptsT = pl.pallas_call(
        _fps_kernel,
        grid=(b,),
        in_specs=[pl.BlockSpec((1, 3, _N), lambda i: (i, 0, 0))],
        out_specs=pl.BlockSpec((1, 3, _NQ), lambda i: (i, 0, 0)),
        out_shape=jax.ShapeDtypeStruct((b, 3, _NQ), jnp.float32),
    )(xyzT)

    knnT = pl.pallas_call(
        _knn_kernel,
        grid=(b, _NQ // _QB),
        in_specs=[pl.BlockSpec((1, 3, _N), lambda i, j: (i, 0, 0)),
                  pl.BlockSpec((1, 3, _QB), lambda i, j: (i, 0, j))],
        out_specs=pl.BlockSpec((1, _K, _QB), lambda i, j: (i, 0, j)),
        out_shape=jax.ShapeDtypeStruct((b, _K, _NQ), jnp.int32),
    )(xyzT, ptsT)

    idx = jnp.transpose(knnT, (0, 2, 1)).astype(jnp.int64)
    pts = jnp.transpose(ptsT, (0, 2, 1))
    return (idx, pts)


# speedup vs baseline: 10.3300x; 1.2167x over previous
import jax
import jax.numpy as jnp
from jax import lax
from jax.experimental import pallas as pl

_N = 8192
_NQ = 1024
_K = 16
_QB = 128
_B = 4


def _fps_kernel(xyz_ref, ptsT_ref):
    iota_n = (lax.broadcasted_iota(jnp.int32, (64, 128), 0) * 128
              + lax.broadcasted_iota(jnp.int32, (64, 128), 1))
    iota_q = (lax.broadcasted_iota(jnp.int32, (8, 128), 0) * 128
              + lax.broadcasted_iota(jnp.int32, (8, 128), 1))
    lane = lax.broadcasted_iota(jnp.int32, (1, 128), 1)

    xs = [xyz_ref[b, 0] for b in range(_B)]
    ys = [xyz_ref[b, 1] for b in range(_B)]
    zs = [xyz_ref[b, 2] for b in range(_B)]

    def body(i, carry):
        qmask = iota_q == i
        out = []
        for b in range(_B):
            dists, far, cxv, cyv, czv = carry[b]
            r = far // 128
            c = far % 128
            cmask = lane == c
            fx = jnp.sum(jnp.where(cmask, xyz_ref[b, 0, pl.ds(r, 1), :], 0.0))
            fy = jnp.sum(jnp.where(cmask, xyz_ref[b, 1, pl.ds(r, 1), :], 0.0))
            fz = jnp.sum(jnp.where(cmask, xyz_ref[b, 2, pl.ds(r, 1), :], 0.0))
            cxv = jnp.where(qmask, fx, cxv)
            cyv = jnp.where(qmask, fy, cyv)
            czv = jnp.where(qmask, fz, czv)
            dx = xs[b] - fx
            dy = ys[b] - fy
            dz = zs[b] - fz
            d = dx * dx + dy * dy + dz * dz
            dists = jnp.minimum(dists, d)
            m = jnp.max(dists)
            far = jnp.min(jnp.where(dists == m, iota_n, jnp.int32(2**30)))
            out.append((dists, far, cxv, cyv, czv))
        return tuple(out)

    init1 = (jnp.full((64, 128), 1e10, dtype=jnp.float32),
             jnp.int32(0),
             jnp.zeros((8, 128), dtype=jnp.float32),
             jnp.zeros((8, 128), dtype=jnp.float32),
             jnp.zeros((8, 128), dtype=jnp.float32))
    final = lax.fori_loop(0, _NQ, body, (init1,) * _B)
    for b in range(_B):
        _, _, cxv, cyv, czv = final[b]
        ptsT_ref[b, 0, :] = cxv.reshape(_NQ)
        ptsT_ref[b, 1, :] = cyv.reshape(_NQ)
        ptsT_ref[b, 2, :] = czv.reshape(_NQ)


def _knn_kernel(xyz_ref, ptsT_ref, out_ref):
    xs = xyz_ref[0, 0, :][None, :]
    ys = xyz_ref[0, 1, :][None, :]
    zs = xyz_ref[0, 2, :][None, :]
    qx = ptsT_ref[0, 0, :][:, None]
    qy = ptsT_ref[0, 1, :][:, None]
    qz = ptsT_ref[0, 2, :][:, None]
    x_sq = xs * xs + ys * ys + zs * zs
    q_sq = qx * qx + qy * qy + qz * qz
    bxs = xs.astype(jnp.bfloat16).astype(jnp.float32)
    bys = ys.astype(jnp.bfloat16).astype(jnp.float32)
    bzs = zs.astype(jnp.bfloat16).astype(jnp.float32)
    bqx = qx.astype(jnp.bfloat16).astype(jnp.float32)
    bqy = qy.astype(jnp.bfloat16).astype(jnp.float32)
    bqz = qz.astype(jnp.bfloat16).astype(jnp.float32)
    dot = bqx * bxs + bqy * bys + bqz * bzs
    d2 = (q_sq + x_sq) - 2.0 * dot
    iota_n = lax.broadcasted_iota(jnp.int32, (_QB, _N), 1)
    for k in range(_K):
        i_k = jnp.argmin(d2, axis=1).astype(jnp.int32)
        out_ref[0, k, :] = i_k
        d2 = jnp.where(iota_n == i_k[:, None], jnp.float32(jnp.inf), d2)


@jax.jit
def kernel(xyz):
    b = xyz.shape[0]
    xyzT = jnp.transpose(xyz, (0, 2, 1))
    xyzR = jnp.reshape(xyzT, (b, 3, 64, 128))

    ptsT = pl.pallas_call(
        _fps_kernel,
        grid=(1,),
        in_specs=[pl.BlockSpec((b, 3, 64, 128), lambda i: (0, 0, 0, 0))],
        out_specs=pl.BlockSpec((b, 3, _NQ), lambda i: (0, 0, 0)),
        out_shape=jax.ShapeDtypeStruct((b, 3, _NQ), jnp.float32),
    )(xyzR)

    knnT = pl.pallas_call(
        _knn_kernel,
        grid=(b, _NQ // _QB),
        in_specs=[pl.BlockSpec((1, 3, _N), lambda i, j: (i, 0, 0)),
                  pl.BlockSpec((1, 3, _QB), lambda i, j: (i, 0, j))],
        out_specs=pl.BlockSpec((1, _K, _QB), lambda i, j: (i, 0, j)),
        out_shape=jax.ShapeDtypeStruct((b, _K, _NQ), jnp.int32),
    )(xyzT, ptsT)

    idx = jnp.transpose(knnT, (0, 2, 1)).astype(jnp.int64)
    pts = jnp.transpose(ptsT, (0, 2, 1))
    return (idx, pts)


# speedup vs baseline: 17.6975x; 1.7132x over previous
import jax
import jax.numpy as jnp
from jax import lax
from jax.experimental import pallas as pl

_N = 8192
_NQ = 1024
_K = 16
_QB = 128
_B = 4


def _fps_kernel(xyzB_ref, ptsB_ref):
    xs = xyzB_ref[0]
    ys = xyzB_ref[1]
    zs = xyzB_ref[2]
    iota_n = lax.broadcasted_iota(jnp.int32, (_B, _N), 1)
    iota_q = lax.broadcasted_iota(jnp.int32, (_B, _NQ), 1)

    def body(i, carry):
        dists, far, cxv, cyv, czv = carry
        nmask = iota_n == far
        fx = jnp.sum(jnp.where(nmask, xs, 0.0), axis=1, keepdims=True)
        fy = jnp.sum(jnp.where(nmask, ys, 0.0), axis=1, keepdims=True)
        fz = jnp.sum(jnp.where(nmask, zs, 0.0), axis=1, keepdims=True)
        qmask = iota_q == i
        cxv = jnp.where(qmask, fx, cxv)
        cyv = jnp.where(qmask, fy, cyv)
        czv = jnp.where(qmask, fz, czv)
        dx = xs - fx
        dy = ys - fy
        dz = zs - fz
        d = dx * dx + dy * dy + dz * dz
        dists = jnp.minimum(dists, d)
        m = jnp.max(dists, axis=1, keepdims=True)
        far = jnp.min(jnp.where(dists == m, iota_n, jnp.int32(2**30)),
                      axis=1, keepdims=True)
        return dists, far, cxv, cyv, czv

    init = (jnp.full((_B, _N), 1e10, dtype=jnp.float32),
            jnp.zeros((_B, 1), dtype=jnp.int32),
            jnp.zeros((_B, _NQ), dtype=jnp.float32),
            jnp.zeros((_B, _NQ), dtype=jnp.float32),
            jnp.zeros((_B, _NQ), dtype=jnp.float32))
    _, _, cxv, cyv, czv = lax.fori_loop(0, _NQ, body, init)
    ptsB_ref[0] = cxv
    ptsB_ref[1] = cyv
    ptsB_ref[2] = czv


def _knn_kernel(xyz_ref, pts_ref, out_ref):
    xs = xyz_ref[0, 0, :][None, :]
    ys = xyz_ref[0, 1, :][None, :]
    zs = xyz_ref[0, 2, :][None, :]
    qx = pts_ref[0, 0, :][:, None]
    qy = pts_ref[0, 1, :][:, None]
    qz = pts_ref[0, 2, :][:, None]
    x_sq = xs * xs + ys * ys + zs * zs
    q_sq = qx * qx + qy * qy + qz * qz
    bxs = xs.astype(jnp.bfloat16).astype(jnp.float32)
    bys = ys.astype(jnp.bfloat16).astype(jnp.float32)
    bzs = zs.astype(jnp.bfloat16).astype(jnp.float32)
    bqx = qx.astype(jnp.bfloat16).astype(jnp.float32)
    bqy = qy.astype(jnp.bfloat16).astype(jnp.float32)
    bqz = qz.astype(jnp.bfloat16).astype(jnp.float32)
    dot = bqx * bxs + bqy * bys + bqz * bzs
    d2 = (q_sq + x_sq) - 2.0 * dot
    iota_n = lax.broadcasted_iota(jnp.int32, (_QB, _N), 1)
    i_prev = None
    for k in range(_K):
        if k > 0:
            d2 = jnp.where(iota_n == i_prev[:, None], jnp.float32(jnp.inf), d2)
        m = jnp.min(d2, axis=1, keepdims=True)
        i_k = jnp.min(jnp.where(d2 == m, iota_n, jnp.int32(2**30)), axis=1)
        out_ref[0, k, :] = i_k
        i_prev = i_k


@jax.jit
def kernel(xyz):
    b = xyz.shape[0]
    xyzB = jnp.transpose(xyz, (2, 0, 1))

    ptsB = pl.pallas_call(
        _fps_kernel,
        grid=(1,),
        in_specs=[pl.BlockSpec((3, b, _N), lambda i: (0, 0, 0))],
        out_specs=pl.BlockSpec((3, b, _NQ), lambda i: (0, 0, 0)),
        out_shape=jax.ShapeDtypeStruct((3, b, _NQ), jnp.float32),
    )(xyzB)

    xyzT = jnp.transpose(xyz, (0, 2, 1))
    ptsT = jnp.transpose(ptsB, (1, 0, 2))

    knnT = pl.pallas_call(
        _knn_kernel,
        grid=(b, _NQ // _QB),
        in_specs=[pl.BlockSpec((1, 3, _N), lambda i, j: (i, 0, 0)),
                  pl.BlockSpec((1, 3, _QB), lambda i, j: (i, 0, j))],
        out_specs=pl.BlockSpec((1, _K, _QB), lambda i, j: (i, 0, j)),
        out_shape=jax.ShapeDtypeStruct((b, _K, _NQ), jnp.int32),
    )(xyzT, ptsT)

    idx = jnp.transpose(knnT, (0, 2, 1)).astype(jnp.int64)
    pts = jnp.transpose(ptsB, (1, 2, 0))
    return (idx, pts)
